# baseline (speedup 1.0000x reference)
import functools

import jax
import jax.numpy as jnp
from jax import lax
from jax.experimental import pallas as pl
from jax.experimental.pallas import tpu as pltpu
from jax.experimental.pallas import tpu_sc as plsc

_LANES = 16


@functools.lru_cache(maxsize=None)
def _sc_gather_fn(B: int):
    info = plsc.get_sparse_core_info()
    nc, ns = info.num_cores, info.num_subcores
    nw = nc * ns
    per_w = B // nw
    assert per_w % _LANES == 0 and B % nw == 0

    mesh = plsc.VectorSubcoreMesh(core_axis_name="c", subcore_axis_name="s")

    @functools.partial(
        pl.kernel,
        out_type=jax.ShapeDtypeStruct((B,), jnp.float32),
        mesh=mesh,
        compiler_params=pltpu.CompilerParams(needs_layout_passes=False),
        scratch_types=[
            pltpu.VMEM((_LANES,), jnp.float32),
            pltpu.VMEM((per_w,), jnp.int32),
            pltpu.VMEM((per_w,), jnp.float32),
        ],
    )
    def sc_gather(table_hbm, xs_hbm, g_hbm, tab_v, idx_v, g_v):
        wid = lax.axis_index("s") * nc + lax.axis_index("c")
        base = wid * per_w
        pltpu.sync_copy(table_hbm, tab_v)
        pltpu.sync_copy(xs_hbm.at[pl.ds(base, per_w)], idx_v)
        for i in range(per_w // _LANES):
            sl = pl.ds(i * _LANES, _LANES)
            g_v[sl] = plsc.load_gather(tab_v, [idx_v[sl]])
        pltpu.sync_copy(g_v, g_hbm.at[pl.ds(base, per_w)])

    return sc_gather


def _tc_body(g_ref, h_ref, w_ref, b_ref, o_ref):
    a = h_ref[...] * w_ref[0, 0] + b_ref[0, 0]
    o_ref[...] = a + g_ref[...]


@functools.lru_cache(maxsize=None)
def _tc_outer_fn(B: int, TI: int):
    grid = (B // TI,)
    return pl.pallas_call(
        _tc_body,
        grid=grid,
        in_specs=[
            pl.BlockSpec((1, B), lambda i: (0, 0)),
            pl.BlockSpec((TI, 1), lambda i: (i, 0)),
            pl.BlockSpec((1, 1), lambda i: (0, 0)),
            pl.BlockSpec((1, 1), lambda i: (0, 0)),
        ],
        out_specs=pl.BlockSpec((TI, B), lambda i: (i, 0)),
        out_shape=jax.ShapeDtypeStruct((B, B), jnp.float32),
    )


def kernel(xs, hs, emb_weight, lin_w, lin_b):
    B = xs.shape[1]
    xs_flat = xs.reshape(B).astype(jnp.int32)
    hs_col = hs.reshape(B, 1)
    n_rows = emb_weight.shape[0]
    table = jnp.pad(emb_weight.reshape(n_rows), (0, _LANES - n_rows))

    g = _sc_gather_fn(B)(table, xs_flat)
    out = _tc_outer_fn(B, 512)(g.reshape(1, B), hs_col,
                               lin_w.reshape(1, 1), lin_b.reshape(1, 1))
    return out.reshape(1, B, B)


# speedup vs baseline: 1.5839x; 1.5839x over previous
import functools

import jax
import jax.numpy as jnp
from jax import lax
from jax.experimental import pallas as pl
from jax.experimental.pallas import tpu as pltpu
from jax.experimental.pallas import tpu_sc as plsc

_LANES = 16


@functools.lru_cache(maxsize=None)
def _sc_gather_fn(B: int):
    info = plsc.get_sparse_core_info()
    nc, ns = info.num_cores, info.num_subcores
    nw = nc * ns
    per_w = B // nw
    assert per_w % _LANES == 0 and B % nw == 0

    mesh = plsc.VectorSubcoreMesh(core_axis_name="c", subcore_axis_name="s")

    @functools.partial(
        pl.kernel,
        out_type=jax.ShapeDtypeStruct((B,), jnp.float32),
        mesh=mesh,
        compiler_params=pltpu.CompilerParams(needs_layout_passes=False),
        scratch_types=[
            pltpu.VMEM((_LANES,), jnp.float32),
            pltpu.VMEM((per_w,), jnp.int32),
            pltpu.VMEM((per_w,), jnp.float32),
        ],
    )
    def sc_gather(table_hbm, xs_hbm, g_hbm, tab_v, idx_v, g_v):
        wid = lax.axis_index("s") * nc + lax.axis_index("c")
        base = wid * per_w
        pltpu.sync_copy(table_hbm, tab_v)
        pltpu.sync_copy(xs_hbm.at[pl.ds(base, per_w)], idx_v)
        for i in range(per_w // _LANES):
            sl = pl.ds(i * _LANES, _LANES)
            g_v[sl] = plsc.load_gather(tab_v, [idx_v[sl]])
        pltpu.sync_copy(g_v, g_hbm.at[pl.ds(base, per_w)])

    return sc_gather


def _tc_body(g_ref, h_ref, w_ref, b_ref, o_ref):
    a = h_ref[...] * w_ref[0, 0] + b_ref[0, 0]
    o_ref[...] = a + g_ref[...]


@functools.lru_cache(maxsize=None)
def _tc_outer_fn(B: int, TI: int):
    grid = (B // TI,)
    return pl.pallas_call(
        _tc_body,
        grid=grid,
        in_specs=[
            pl.BlockSpec((1, B), lambda i: (0, 0)),
            pl.BlockSpec((TI, 1), lambda i: (i, 0)),
            pl.BlockSpec((1, 1), lambda i: (0, 0)),
            pl.BlockSpec((1, 1), lambda i: (0, 0)),
        ],
        out_specs=pl.BlockSpec((TI, B), lambda i: (i, 0)),
        out_shape=jax.ShapeDtypeStruct((B, B), jnp.float32),
    )


def _tc_fused_body(x_ref, h_ref, t_ref, w_ref, b_ref, o_ref):
    x = x_ref[...]
    g = jnp.where(x == 1, t_ref[0, 1], t_ref[0, 0])
    g = jnp.where(x == 2, t_ref[0, 2], g)
    g = jnp.where(x == 3, t_ref[0, 3], g)
    a = h_ref[...] * w_ref[0, 0] + b_ref[0, 0]
    o_ref[...] = a + g


@functools.lru_cache(maxsize=None)
def _tc_fused_fn(B: int, TI: int):
    grid = (B // TI,)
    return pl.pallas_call(
        _tc_fused_body,
        grid=grid,
        in_specs=[
            pl.BlockSpec((1, B), lambda i: (0, 0)),
            pl.BlockSpec((TI, 1), lambda i: (i, 0)),
            pl.BlockSpec((1, _LANES), lambda i: (0, 0)),
            pl.BlockSpec((1, 1), lambda i: (0, 0)),
            pl.BlockSpec((1, 1), lambda i: (0, 0)),
        ],
        out_specs=pl.BlockSpec((TI, B), lambda i: (i, 0)),
        out_shape=jax.ShapeDtypeStruct((B, B), jnp.float32),
    )


def kernel(xs, hs, emb_weight, lin_w, lin_b):
    B = xs.shape[1]
    xs_flat = xs.reshape(B).astype(jnp.int32)
    hs_col = hs.reshape(B, 1)
    n_rows = emb_weight.shape[0]
    table = jnp.pad(emb_weight.reshape(n_rows), (0, _LANES - n_rows))

    out = _tc_fused_fn(B, 512)(xs_flat.reshape(1, B), hs_col,
                               table.reshape(1, _LANES),
                               lin_w.reshape(1, 1), lin_b.reshape(1, 1))
    return out.reshape(1, B, B)
